# baseline (speedup 1.0000x reference)
import functools

import jax
import jax.numpy as jnp
from jax import lax
from jax.experimental import pallas as pl
from jax.experimental.pallas import tpu as pltpu
from jax.experimental.pallas import tpu_sc as plsc

_SQRT_HALF = 0.7071067811865476


def _gelu(t):
    return 0.5 * t * (1.0 + lax.erf(t * _SQRT_HALF))


def _prep_body(inp_ref, gi_ref, go_ref, pw0_ref, pb0_ref, pw1_ref, pb1_ref,
               kw0a_ref, kw0b_ref, kw0f_ref, kb0_ref,
               x_ref, u_ref, g_ref):
    h = jnp.dot(inp_ref[...], pw0_ref[...], preferred_element_type=jnp.float32) + pb0_ref[...]
    x = jnp.dot(_gelu(h), pw1_ref[...], preferred_element_type=jnp.float32) + pb1_ref[...]
    x_ref[...] = x
    u_ref[pl.ds(0, x.shape[0]), :] = (
        jnp.dot(x, kw0f_ref[...], preferred_element_type=jnp.float32)
        + jnp.dot(gi_ref[...], kw0a_ref[...], preferred_element_type=jnp.float32))
    g_ref[...] = (jnp.dot(go_ref[...], kw0b_ref[...], preferred_element_type=jnp.float32)
                  + kb0_ref[...])


def _post_body(rows_ref, g_ref, x_ref, kw1_ref, kb1_ref, lng_ref, lnb_ref, out_ref):
    d = g_ref.shape[-1]
    rows = rows_ref[...][:, :, :d]
    hidden = _gelu(rows + g_ref[...][:, None, :])
    s = jnp.mean(hidden, axis=1)
    o = (jnp.dot(s, kw1_ref[...], preferred_element_type=jnp.float32)
         + kb1_ref[...] + x_ref[...])
    mu = jnp.mean(o, axis=-1, keepdims=True)
    dv = o - mu
    var = jnp.mean(dv * dv, axis=-1, keepdims=True)
    out_ref[...] = dv * lax.rsqrt(var + 1e-5) * lng_ref[...] + lnb_ref[...]


def _sc_gather(table, idx, n_out, d, e0=0):
    nv, dp = table.shape
    info = plsc.get_sparse_core_info()
    nw = info.num_cores * info.num_subcores
    per_w = n_out // nw
    w = 40
    n_chunks = per_w // w
    mesh = plsc.VectorSubcoreMesh(core_axis_name="c", subcore_axis_name="s")

    @functools.partial(
        pl.kernel,
        out_type=jax.ShapeDtypeStruct((n_out, dp), jnp.float32),
        mesh=mesh,
        scratch_types=[
            pltpu.VMEM_SHARED((nv, dp), jnp.float32),
            pltpu.VMEM((w,), jnp.int32),
            pltpu.VMEM((w,), jnp.int32),
            pltpu.VMEM((w, dp), jnp.float32),
            pltpu.VMEM((w, dp), jnp.float32),
            pltpu.SemaphoreType.DMA,
            pltpu.SemaphoreType.DMA,
            pltpu.SemaphoreType.DMA,
            pltpu.SemaphoreType.DMA,
        ],
    )
    def k(table_hbm, idx_hbm, out_hbm, tab_sh, idx_va, idx_vb, rows_a, rows_b,
          sem_i, sem_g, sem_a, sem_b):
        cid = lax.axis_index("c")
        sid = lax.axis_index("s")
        wid = sid * info.num_cores + cid
        stage_rows = nv // info.num_subcores
        pltpu.async_copy(table_hbm.at[pl.ds(sid * stage_rows, stage_rows)],
                         tab_sh.at[pl.ds(sid * stage_rows, stage_rows)], sem_i).wait()
        plsc.subcore_barrier()
        base0 = e0 + wid * per_w
        pltpu.async_copy(idx_hbm.at[pl.ds(base0, w)], idx_va, sem_i).wait()

        @pl.loop(0, n_chunks, step=2)
        def _(ci):
            for j, rows_v, sem_o, idx_cur, idx_nxt in (
                    (0, rows_a, sem_a, idx_va, idx_vb),
                    (1, rows_b, sem_b, idx_vb, idx_va)):
                cur = ci + j

                @pl.when(cur < n_chunks)
                def _():
                    base = wid * per_w + cur * w
                    gbase = base0 + cur * w
                    nxt = gbase + w

                    @pl.when(cur + 1 < n_chunks)
                    def _():
                        pltpu.async_copy(idx_hbm.at[pl.ds(nxt, w)], idx_nxt, sem_i)

                    @pl.when(cur >= 2)
                    def _():
                        pltpu.make_async_copy(rows_v, out_hbm.at[pl.ds(base0, w)],
                                              sem_o).wait()

                    pltpu.async_copy(tab_sh.at[idx_cur], rows_v, sem_g).wait()
                    pltpu.async_copy(rows_v, out_hbm.at[pl.ds(base, w)], sem_o)

                    @pl.when(cur + 1 < n_chunks)
                    def _():
                        pltpu.make_async_copy(idx_hbm.at[pl.ds(nxt, w)],
                                              idx_nxt, sem_i).wait()

        pltpu.make_async_copy(rows_a, out_hbm.at[pl.ds(wid * per_w, w)], sem_a).wait()
        pltpu.make_async_copy(rows_b, out_hbm.at[pl.ds(wid * per_w, w)], sem_b).wait()

    return k(table, idx)


def kernel(inp, input_grid, output_grid, nbr_idx, P_W0, P_b0, P_W1, P_b1,
           K_W0, K_b0, K_W1, K_b1, ln_g, ln_b):
    B, N, _ = inp.shape
    D = P_W1.shape[1]
    K = nbr_idx.shape[1]
    nd = input_grid.shape[1]
    NP = 10240

    inp2 = inp.reshape(N, -1)
    DP = 128
    kw0a = jnp.pad(K_W0[:nd], ((0, 0), (0, DP - D)))
    kw0b = K_W0[nd:2 * nd]
    kw0f = jnp.pad(K_W0[2 * nd:], ((0, 0), (0, DP - D)))

    x, u, g = pl.pallas_call(
        _prep_body,
        out_shape=[jax.ShapeDtypeStruct((N, D), jnp.float32),
                   jax.ShapeDtypeStruct((NP, DP), jnp.float32),
                   jax.ShapeDtypeStruct((N, D), jnp.float32)],
    )(inp2, input_grid, output_grid, P_W0, P_b0.reshape(1, -1), P_W1,
      P_b1.reshape(1, -1), kw0a, kw0b, kw0f, K_b0.reshape(1, -1))

    idx_all = nbr_idx.astype(jnp.int32).reshape(-1)
    CH = 2
    NC = N // CH
    EC = NC * K
    NB = 200
    wts = (K_W1, K_b1.reshape(1, -1), ln_g.reshape(1, -1), ln_b.reshape(1, -1))
    outs = []
    for c in range(CH):
        rows = _sc_gather(u, idx_all, EC, D, e0=c * EC).reshape(NC, K, DP)
        base = c * (NC // NB)
        out_c = pl.pallas_call(
            _post_body,
            grid=(NC // NB,),
            in_specs=[
                pl.BlockSpec((NB, K, DP), lambda i: (i, 0, 0)),
                pl.BlockSpec((NB, D), lambda i, b=base: (b + i, 0)),
                pl.BlockSpec((NB, D), lambda i, b=base: (b + i, 0)),
                pl.BlockSpec((D, D), lambda i: (0, 0)),
                pl.BlockSpec((1, D), lambda i: (0, 0)),
                pl.BlockSpec((1, D), lambda i: (0, 0)),
                pl.BlockSpec((1, D), lambda i: (0, 0)),
            ],
            out_specs=pl.BlockSpec((NB, D), lambda i: (i, 0)),
            out_shape=jax.ShapeDtypeStruct((NC, D), jnp.float32),
        )(rows, g, x, *wts)
        outs.append(out_c)

    return jnp.concatenate(outs, axis=0).reshape(B, N, D)


# speedup vs baseline: 1.1958x; 1.1958x over previous
import functools

import jax
import jax.numpy as jnp
from jax import lax
from jax.experimental import pallas as pl
from jax.experimental.pallas import tpu as pltpu
from jax.experimental.pallas import tpu_sc as plsc

_SQRT_HALF = 0.7071067811865476


def _gelu(t):
    return 0.5 * t * (1.0 + lax.erf(t * _SQRT_HALF))


def _prep_body(inp_ref, gi_ref, go_ref, pw0_ref, pb0_ref, pw1_ref, pb1_ref,
               kw0a_ref, kw0b_ref, kw0f_ref, kb0_ref,
               x_ref, u_ref, g_ref):
    h = jnp.dot(inp_ref[...], pw0_ref[...], preferred_element_type=jnp.float32) + pb0_ref[...]
    x = jnp.dot(_gelu(h), pw1_ref[...], preferred_element_type=jnp.float32) + pb1_ref[...]
    x_ref[...] = x
    u_ref[pl.ds(0, x.shape[0]), :] = (
        jnp.dot(x, kw0f_ref[...], preferred_element_type=jnp.float32)
        + jnp.dot(gi_ref[...], kw0a_ref[...], preferred_element_type=jnp.float32))
    g_ref[...] = (jnp.dot(go_ref[...], kw0b_ref[...], preferred_element_type=jnp.float32)
                  + kb0_ref[...])


def _post_body(rows_ref, g_ref, x_ref, kw1_ref, kb1_ref, lng_ref, lnb_ref, out_ref):
    d = x_ref.shape[-1]
    k = rows_ref.shape[1]
    hidden = _gelu(rows_ref[...] + g_ref[...][:, None, :])
    s = jnp.sum(hidden, axis=1)[:, :d] * (1.0 / k)
    o = (jnp.dot(s, kw1_ref[...], preferred_element_type=jnp.float32)
         + kb1_ref[...] + x_ref[...])
    mu = jnp.mean(o, axis=-1, keepdims=True)
    dv = o - mu
    var = jnp.mean(dv * dv, axis=-1, keepdims=True)
    out_ref[...] = (dv * lax.rsqrt(var + 1e-5) * lng_ref[...] + lnb_ref[...])[None]


def _sc_gather(table, idx, n_out, d):
    nv, dp = table.shape
    info = plsc.get_sparse_core_info()
    nw = info.num_cores * info.num_subcores
    per_w = n_out // nw
    w = 80
    n_chunks = per_w // w
    mesh = plsc.VectorSubcoreMesh(core_axis_name="c", subcore_axis_name="s")

    @functools.partial(
        pl.kernel,
        out_type=jax.ShapeDtypeStruct((n_out, dp), jnp.float32),
        mesh=mesh,
        scratch_types=[
            pltpu.VMEM_SHARED((nv, dp), jnp.float32),
            pltpu.VMEM((w,), jnp.int32),
            pltpu.VMEM((w,), jnp.int32),
            pltpu.VMEM((w, dp), jnp.float32),
            pltpu.VMEM((w, dp), jnp.float32),
            pltpu.SemaphoreType.DMA,
            pltpu.SemaphoreType.DMA,
            pltpu.SemaphoreType.DMA,
            pltpu.SemaphoreType.DMA,
        ],
    )
    def k(table_hbm, idx_hbm, out_hbm, tab_sh, idx_va, idx_vb, rows_a, rows_b,
          sem_i, sem_g, sem_a, sem_b):
        cid = lax.axis_index("c")
        sid = lax.axis_index("s")
        wid = sid * info.num_cores + cid
        stage_rows = nv // info.num_subcores
        pltpu.async_copy(table_hbm.at[pl.ds(sid * stage_rows, stage_rows)],
                         tab_sh.at[pl.ds(sid * stage_rows, stage_rows)], sem_i).wait()
        plsc.subcore_barrier()
        base0 = wid * per_w
        pltpu.async_copy(idx_hbm.at[pl.ds(base0, w)], idx_va, sem_i).wait()

        @pl.loop(0, n_chunks, step=2)
        def _(ci):
            for j, rows_v, sem_o, idx_cur, idx_nxt in (
                    (0, rows_a, sem_a, idx_va, idx_vb),
                    (1, rows_b, sem_b, idx_vb, idx_va)):
                cur = ci + j

                @pl.when(cur < n_chunks)
                def _():
                    base = base0 + cur * w
                    nxt = base + w

                    @pl.when(cur + 1 < n_chunks)
                    def _():
                        pltpu.async_copy(idx_hbm.at[pl.ds(nxt, w)], idx_nxt, sem_i)

                    @pl.when(cur >= 2)
                    def _():
                        pltpu.make_async_copy(rows_v, out_hbm.at[pl.ds(base0, w)],
                                              sem_o).wait()

                    pltpu.async_copy(tab_sh.at[idx_cur], rows_v, sem_g).wait()
                    pltpu.async_copy(rows_v, out_hbm.at[pl.ds(base, w)], sem_o)

                    @pl.when(cur + 1 < n_chunks)
                    def _():
                        pltpu.make_async_copy(idx_hbm.at[pl.ds(nxt, w)],
                                              idx_nxt, sem_i).wait()

        pltpu.make_async_copy(rows_a, out_hbm.at[pl.ds(base0, w)], sem_a).wait()
        pltpu.make_async_copy(rows_b, out_hbm.at[pl.ds(base0, w)], sem_b).wait()

    return k(table, idx)


def kernel(inp, input_grid, output_grid, nbr_idx, P_W0, P_b0, P_W1, P_b1,
           K_W0, K_b0, K_W1, K_b1, ln_g, ln_b):
    B, N, _ = inp.shape
    D = P_W1.shape[1]
    K = nbr_idx.shape[1]
    nd = input_grid.shape[1]
    NP = 10240

    inp2 = inp.reshape(N, -1)
    DP = 128
    kw0a = jnp.pad(K_W0[:nd], ((0, 0), (0, DP - D)))
    kw0b = jnp.pad(K_W0[nd:2 * nd], ((0, 0), (0, DP - D)))
    kw0f = jnp.pad(K_W0[2 * nd:], ((0, 0), (0, DP - D)))

    x, u, g = pl.pallas_call(
        _prep_body,
        out_shape=[jax.ShapeDtypeStruct((N, D), jnp.float32),
                   jax.ShapeDtypeStruct((NP, DP), jnp.float32),
                   jax.ShapeDtypeStruct((N, DP), jnp.float32)],
    )(inp2, input_grid, output_grid, P_W0, P_b0.reshape(1, -1), P_W1,
      P_b1.reshape(1, -1), kw0a, kw0b, kw0f,
      jnp.pad(K_b0, (0, DP - D)).reshape(1, -1))

    idx_all = nbr_idx.astype(jnp.int32).reshape(-1)
    E = N * K
    rows = _sc_gather(u, idx_all, E, D).reshape(N, K, DP)

    NB = 400
    out = pl.pallas_call(
        _post_body,
        grid=(N // NB,),
        in_specs=[
            pl.BlockSpec((NB, K, DP), lambda i: (i, 0, 0)),
            pl.BlockSpec((NB, DP), lambda i: (i, 0)),
            pl.BlockSpec((NB, D), lambda i: (i, 0)),
            pl.BlockSpec((D, D), lambda i: (0, 0)),
            pl.BlockSpec((1, D), lambda i: (0, 0)),
            pl.BlockSpec((1, D), lambda i: (0, 0)),
            pl.BlockSpec((1, D), lambda i: (0, 0)),
        ],
        out_specs=pl.BlockSpec((1, NB, D), lambda i: (0, i, 0)),
        out_shape=jax.ShapeDtypeStruct((B, N, D), jnp.float32),
    )(rows, g, x, K_W1, K_b1.reshape(1, -1), ln_g.reshape(1, -1), ln_b.reshape(1, -1))

    return out


# speedup vs baseline: 1.2302x; 1.0288x over previous
import functools

import jax
import jax.numpy as jnp
from jax import lax
from jax.experimental import pallas as pl
from jax.experimental.pallas import tpu as pltpu
from jax.experimental.pallas import tpu_sc as plsc

_SQRT_HALF = 0.7071067811865476


def _gelu(t):
    return 0.5 * t * (1.0 + lax.erf(t * _SQRT_HALF))


def _prep_body(inp_ref, gi_ref, go_ref, pw0_ref, pb0_ref, pw1_ref, pb1_ref,
               kw0a_ref, kw0b_ref, kw0f_ref, kb0_ref,
               x_ref, u_ref, g_ref):
    h = jnp.dot(inp_ref[...], pw0_ref[...], preferred_element_type=jnp.float32) + pb0_ref[...]
    x = jnp.dot(_gelu(h), pw1_ref[...], preferred_element_type=jnp.float32) + pb1_ref[...]
    x_ref[...] = x
    u_ref[...] = (
        jnp.dot(x, kw0f_ref[...], preferred_element_type=jnp.float32)
        + jnp.dot(gi_ref[...], kw0a_ref[...], preferred_element_type=jnp.float32))
    g_ref[...] = (jnp.dot(go_ref[...], kw0b_ref[...], preferred_element_type=jnp.float32)
                  + kb0_ref[...])


def _post_body(rows_ref, g_ref, x_ref, kw1_ref, kb1_ref, lng_ref, lnb_ref, out_ref):
    d = x_ref.shape[-1]
    k = rows_ref.shape[1]
    tp = rows_ref[...] + g_ref[...][:, None, :]
    q = tp * (1.0 + lax.erf(tp))
    s = jnp.sum(q, axis=1)[:, :d] * (_SQRT_HALF / k)
    o = (jnp.dot(s, kw1_ref[...], preferred_element_type=jnp.float32)
         + kb1_ref[...] + x_ref[...])
    mu = jnp.mean(o, axis=-1, keepdims=True)
    dv = o - mu
    var = jnp.mean(dv * dv, axis=-1, keepdims=True)
    out_ref[...] = (dv * lax.rsqrt(var + 1e-5) * lng_ref[...] + lnb_ref[...])[None]


def _sc_gather(table, idx, n_out, d):
    nv, dp = table.shape
    info = plsc.get_sparse_core_info()
    nw = info.num_cores * info.num_subcores
    per_w = n_out // nw
    w = 80
    n_chunks = per_w // w
    mesh = plsc.VectorSubcoreMesh(core_axis_name="c", subcore_axis_name="s")

    @functools.partial(
        pl.kernel,
        out_type=jax.ShapeDtypeStruct((n_out, dp), jnp.float32),
        mesh=mesh,
        scratch_types=[
            pltpu.VMEM_SHARED((nv, dp), jnp.float32),
            pltpu.VMEM((w,), jnp.int32),
            pltpu.VMEM((w,), jnp.int32),
            pltpu.VMEM((w, dp), jnp.float32),
            pltpu.VMEM((w, dp), jnp.float32),
            pltpu.SemaphoreType.DMA,
            pltpu.SemaphoreType.DMA,
            pltpu.SemaphoreType.DMA,
            pltpu.SemaphoreType.DMA,
        ],
    )
    def k(table_hbm, idx_hbm, out_hbm, tab_sh, idx_va, idx_vb, rows_a, rows_b,
          sem_i, sem_g, sem_a, sem_b):
        cid = lax.axis_index("c")
        sid = lax.axis_index("s")
        wid = sid * info.num_cores + cid
        stage_rows = nv // info.num_subcores
        pltpu.async_copy(table_hbm.at[pl.ds(sid * stage_rows, stage_rows)],
                         tab_sh.at[pl.ds(sid * stage_rows, stage_rows)], sem_i).wait()
        plsc.subcore_barrier()
        base0 = wid * per_w
        pltpu.async_copy(idx_hbm.at[pl.ds(base0, w)], idx_va, sem_i).wait()

        @pl.loop(0, n_chunks, step=2)
        def _(ci):
            for j, rows_v, sem_o, idx_cur, idx_nxt in (
                    (0, rows_a, sem_a, idx_va, idx_vb),
                    (1, rows_b, sem_b, idx_vb, idx_va)):
                cur = ci + j

                @pl.when(cur < n_chunks)
                def _():
                    base = base0 + cur * w
                    nxt = base + w

                    @pl.when(cur + 1 < n_chunks)
                    def _():
                        pltpu.async_copy(idx_hbm.at[pl.ds(nxt, w)], idx_nxt, sem_i)

                    @pl.when(cur >= 2)
                    def _():
                        pltpu.make_async_copy(rows_v, out_hbm.at[pl.ds(base0, w)],
                                              sem_o).wait()

                    pltpu.async_copy(tab_sh.at[idx_cur], rows_v, sem_g).wait()
                    pltpu.async_copy(rows_v, out_hbm.at[pl.ds(base, w)], sem_o)

                    @pl.when(cur + 1 < n_chunks)
                    def _():
                        pltpu.make_async_copy(idx_hbm.at[pl.ds(nxt, w)],
                                              idx_nxt, sem_i).wait()

        pltpu.make_async_copy(rows_a, out_hbm.at[pl.ds(base0, w)], sem_a).wait()
        pltpu.make_async_copy(rows_b, out_hbm.at[pl.ds(base0, w)], sem_b).wait()

    return k(table, idx)


def kernel(inp, input_grid, output_grid, nbr_idx, P_W0, P_b0, P_W1, P_b1,
           K_W0, K_b0, K_W1, K_b1, ln_g, ln_b):
    B, N, _ = inp.shape
    D = P_W1.shape[1]
    K = nbr_idx.shape[1]
    nd = input_grid.shape[1]
    NP = 10240

    inp2 = inp.reshape(N, -1)
    DP = 128
    kw0a = jnp.pad(K_W0[:nd], ((0, 0), (0, DP - D))) * _SQRT_HALF
    kw0b = jnp.pad(K_W0[nd:2 * nd], ((0, 0), (0, DP - D))) * _SQRT_HALF
    kw0f = jnp.pad(K_W0[2 * nd:], ((0, 0), (0, DP - D))) * _SQRT_HALF

    AB = 1280
    IN = inp2.shape[1]
    x, u, g = pl.pallas_call(
        _prep_body,
        grid=(NP // AB,),
        in_specs=[
            pl.BlockSpec((AB, IN), lambda i: (i, 0)),
            pl.BlockSpec((AB, nd), lambda i: (i, 0)),
            pl.BlockSpec((AB, nd), lambda i: (i, 0)),
            pl.BlockSpec((IN, 256), lambda i: (0, 0)),
            pl.BlockSpec((1, 256), lambda i: (0, 0)),
            pl.BlockSpec((256, D), lambda i: (0, 0)),
            pl.BlockSpec((1, D), lambda i: (0, 0)),
            pl.BlockSpec((nd, DP), lambda i: (0, 0)),
            pl.BlockSpec((nd, DP), lambda i: (0, 0)),
            pl.BlockSpec((D, DP), lambda i: (0, 0)),
            pl.BlockSpec((1, DP), lambda i: (0, 0)),
        ],
        out_specs=[pl.BlockSpec((AB, D), lambda i: (i, 0)),
                   pl.BlockSpec((AB, DP), lambda i: (i, 0)),
                   pl.BlockSpec((AB, DP), lambda i: (i, 0))],
        out_shape=[jax.ShapeDtypeStruct((N, D), jnp.float32),
                   jax.ShapeDtypeStruct((NP, DP), jnp.float32),
                   jax.ShapeDtypeStruct((N, DP), jnp.float32)],
    )(inp2, input_grid, output_grid, P_W0, P_b0.reshape(1, -1), P_W1,
      P_b1.reshape(1, -1), kw0a, kw0b, kw0f,
      (jnp.pad(K_b0, (0, DP - D)) * _SQRT_HALF).reshape(1, -1))

    idx_all = nbr_idx.astype(jnp.int32).reshape(-1)
    E = N * K
    rows = _sc_gather(u, idx_all, E, D).reshape(N, K, DP)

    NB = 400
    out = pl.pallas_call(
        _post_body,
        grid=(N // NB,),
        in_specs=[
            pl.BlockSpec((NB, K, DP), lambda i: (i, 0, 0)),
            pl.BlockSpec((NB, DP), lambda i: (i, 0)),
            pl.BlockSpec((NB, D), lambda i: (i, 0)),
            pl.BlockSpec((D, D), lambda i: (0, 0)),
            pl.BlockSpec((1, D), lambda i: (0, 0)),
            pl.BlockSpec((1, D), lambda i: (0, 0)),
            pl.BlockSpec((1, D), lambda i: (0, 0)),
        ],
        out_specs=pl.BlockSpec((1, NB, D), lambda i: (0, i, 0)),
        out_shape=jax.ShapeDtypeStruct((B, N, D), jnp.float32),
    )(rows, g, x, K_W1, K_b1.reshape(1, -1), ln_g.reshape(1, -1), ln_b.reshape(1, -1))

    return out
